# speedup vs baseline: 4.3599x; 4.3599x over previous
import jax
import jax.numpy as jnp
from jax.experimental import pallas as pl

EPS = 1e-06
N_BINS = 256

_ROWS_PER_STEP = 16


def _onehot_body(x_ref, out_ref):
    x = x_ref[...]
    x = jnp.clip(x, EPS, 1.0 - EPS)
    idx = (x * N_BINS).astype(jnp.int32)
    idx = jnp.clip(idx, 0, N_BINS - 1)
    iota = jax.lax.broadcasted_iota(
        jnp.int32, (x.shape[0], x.shape[1], N_BINS), 2)
    out_ref[...] = (iota == idx[..., None]).astype(jnp.float32)


def kernel(proprio):
    b, t, f = proprio.shape
    n = b * t * f
    cols = N_BINS
    rows = n // cols
    x2 = proprio.reshape(rows, cols)
    grid = rows // _ROWS_PER_STEP
    out = pl.pallas_call(
        _onehot_body,
        grid=(grid,),
        in_specs=[pl.BlockSpec((_ROWS_PER_STEP, cols), lambda i: (i, 0))],
        out_specs=pl.BlockSpec((_ROWS_PER_STEP, cols, N_BINS),
                               lambda i: (i, 0, 0)),
        out_shape=jax.ShapeDtypeStruct((rows, cols, N_BINS), jnp.float32),
    )(x2)
    tokens = out.reshape(b, t, f, N_BINS)
    mask = jnp.ones((b, t, f), dtype=bool)
    return tokens, mask
